# Initial kernel scaffold; baseline (speedup 1.0000x reference)
import functools
import jax
import jax.numpy as jnp
from jax import lax
from jax.experimental import pallas as pl
from jax.experimental.pallas import tpu as pltpu
from jax.experimental.pallas import tpu_sc as plsc

HIDK = 64
TDIMK = 64

N_NODES = 10000
N_EDGES = 320000
NB_ROWS = 1000
EB_ROWS = 512
SC_CHUNK = 80
ROWS_PER_TILE = N_NODES // 16
EDGES_PER_TILE = N_EDGES // 32
N_CHUNKS = EDGES_PER_TILE // SC_CHUNK


def _silu(x):
    return x * jax.nn.sigmoid(x)


def _ln2(x, g, b):
    mu = jnp.mean(x, axis=-1, keepdims=True)
    var = jnp.var(x, axis=-1, keepdims=True)
    return (x - mu) / jnp.sqrt(var + 1e-5) * g + b


def _k1_body(theta_ref, batch_ref, emb_ref, wn_ref, bn_ref,
             wt0_ref, bt0_ref, wt1_ref, bt1_ref, wtp_ref, btp_ref,
             wr_ref, wc_ref,
             x_ref, xr_ref, xc_ref):
    th = theta_ref[...]
    m = jnp.max(th, axis=-1, keepdims=True)
    e = jnp.exp(th - m)
    sm = e / jnp.sum(e, axis=-1, keepdims=True)
    x = jnp.dot(sm, wn_ref[...], preferred_element_type=jnp.float32) + bn_ref[...]
    emb = emb_ref[...]
    t0 = _silu(jnp.dot(emb, wt0_ref[...], preferred_element_type=jnp.float32) + bt0_ref[...])
    temb = jnp.dot(t0, wt1_ref[...], preferred_element_type=jnp.float32) + bt1_ref[...]
    temb = jnp.dot(temb, wtp_ref[...], preferred_element_type=jnp.float32) + btp_ref[...]
    bb = batch_ref[...]
    oh = (bb == lax.broadcasted_iota(jnp.int32, (1, 32), 1)).astype(jnp.float32)
    x = x + jnp.dot(oh, temb, preferred_element_type=jnp.float32)
    x_ref[...] = x
    xr_ref[...] = jnp.dot(x, wr_ref[...], preferred_element_type=jnp.float32)
    xc_ref[...] = jnp.dot(x, wc_ref[...], preferred_element_type=jnp.float32)


def _k1_call(theta_pad, batch2, emb, wn_pad, bn, wt0, bt0, wt1, bt1, wtp, btp, wr, wc):
    n = theta_pad.shape[0]
    grid = (n // NB_ROWS,)
    full = lambda shp: pl.BlockSpec(shp, lambda i: (0,) * len(shp))
    return pl.pallas_call(
        _k1_body,
        grid=grid,
        in_specs=[
            pl.BlockSpec((NB_ROWS, 128), lambda i: (i, 0)),
            pl.BlockSpec((NB_ROWS, 1), lambda i: (i, 0)),
            full((32, TDIMK)),
            full((128, HIDK)), full((1, HIDK)),
            full((TDIMK, TDIMK)), full((1, TDIMK)),
            full((TDIMK, TDIMK)), full((1, TDIMK)),
            full((TDIMK, HIDK)), full((1, HIDK)),
            full((HIDK, HIDK)), full((HIDK, HIDK)),
        ],
        out_specs=[pl.BlockSpec((NB_ROWS, HIDK), lambda i: (i, 0))] * 3,
        out_shape=[jax.ShapeDtypeStruct((n, HIDK), jnp.float32)] * 3,
    )(theta_pad, batch2, emb, wn_pad, bn, wt0, bt0, wt1, bt1, wtp, btp, wr, wc)


def _pre_kernel_body(pos_hbm, row_hbm, col_hbm, ev_out, deg_out,
                     pos_sh, deg_sh, idx_r, idx_c, pr, pc, evb, onesb, stage,
                     sem_a, sem_b):
    c = lax.axis_index("c")
    s = lax.axis_index("s")
    wid = c * 16 + s
    r0 = s * ROWS_PER_TILE

    def zbody(i, _):
        stage[i, :] = jnp.zeros((16,), jnp.float32)
        return 0
    lax.fori_loop(0, ROWS_PER_TILE, zbody, 0)
    pltpu.sync_copy(stage, deg_sh.at[pl.ds(r0, ROWS_PER_TILE)])
    pltpu.sync_copy(pos_hbm.at[pl.ds(r0, ROWS_PER_TILE)], stage)
    pltpu.sync_copy(stage, pos_sh.at[pl.ds(r0, ROWS_PER_TILE)])

    def obody(i, _):
        onesb[i, :] = jnp.full((16,), 1.0, jnp.float32)
        return 0
    lax.fori_loop(0, SC_CHUNK, obody, 0)
    plsc.subcore_barrier()

    base = wid * EDGES_PER_TILE

    def chunk(k, _):
        e0 = base + k * SC_CHUNK
        pltpu.sync_copy(row_hbm.at[pl.ds(e0, SC_CHUNK)], idx_r)
        pltpu.sync_copy(col_hbm.at[pl.ds(e0, SC_CHUNK)], idx_c)
        cp_r = pltpu.async_copy(pos_sh.at[idx_r], pr, sem_a)
        cp_c = pltpu.async_copy(pos_sh.at[idx_c], pc, sem_b)
        cp_r.wait()
        cp_c.wait()

        def ebody(i, _):
            evb[i, :] = pc[i, :] - pr[i, :]
            return 0
        lax.fori_loop(0, SC_CHUNK, ebody, 0)
        pltpu.sync_copy(evb, ev_out.at[pl.ds(e0, SC_CHUNK)])
        pltpu.sync_copy(onesb, deg_sh.at[idx_r], add=True)
        return 0

    lax.fori_loop(0, N_CHUNKS, chunk, 0)
    plsc.subcore_barrier()
    pltpu.sync_copy(deg_sh.at[pl.ds(r0, ROWS_PER_TILE)], stage)
    pltpu.sync_copy(stage, deg_out.at[c, pl.ds(r0, ROWS_PER_TILE)])


def _pre_call(pos_pad, row, col):
    mesh = plsc.VectorSubcoreMesh(core_axis_name="c", subcore_axis_name="s")
    kfn = functools.partial(
        pl.kernel,
        mesh=mesh,
        out_type=[
            jax.ShapeDtypeStruct((N_EDGES, 16), jnp.float32),
            jax.ShapeDtypeStruct((2, N_NODES, 16), jnp.float32),
        ],
        scratch_types=[
            pltpu.VMEM_SHARED((N_NODES, 16), jnp.float32),
            pltpu.VMEM_SHARED((N_NODES, 16), jnp.float32),
            pltpu.VMEM((SC_CHUNK,), jnp.int32),
            pltpu.VMEM((SC_CHUNK,), jnp.int32),
            pltpu.VMEM((SC_CHUNK, 16), jnp.float32),
            pltpu.VMEM((SC_CHUNK, 16), jnp.float32),
            pltpu.VMEM((SC_CHUNK, 16), jnp.float32),
            pltpu.VMEM((SC_CHUNK, 16), jnp.float32),
            pltpu.VMEM((ROWS_PER_TILE, 16), jnp.float32),
            pltpu.SemaphoreType.DMA,
            pltpu.SemaphoreType.DMA,
        ],
    )
    return kfn(_pre_kernel_body)(pos_pad, row, col)


def _k2_body(ev_ref, wd0_ref, bd0_ref, wd1_ref, bd1_ref,
             wr0_ref, br0_ref, wr1_ref, br1_ref,
             g_ref, b_ref, weall_ref, b0all_ref,
             ef0_ref, ef1_ref, ef2_ref, ef3_ref):
    ev = ev_ref[...][:, 0:3]
    d2 = jnp.sum(ev * ev, axis=-1, keepdims=True)
    d = jnp.sqrt(d2)
    dirv = ev / (d + 1e-8)
    h = _silu(d * wd0_ref[...] + bd0_ref[...])
    df = jnp.dot(h, wd1_ref[...], preferred_element_type=jnp.float32) + bd1_ref[...]
    wr0 = wr0_ref[...]
    rh = _silu(dirv[:, 0:1] * wr0[0:1, :] + dirv[:, 1:2] * wr0[1:2, :]
               + dirv[:, 2:3] * wr0[2:3, :] + br0_ref[...])
    rf = jnp.dot(rh, wr1_ref[...], preferred_element_type=jnp.float32) + br1_ref[...]
    ef = jnp.concatenate([df, rf], axis=-1)
    ef = _ln2(ef, g_ref[...], b_ref[...])
    weall = weall_ref[...]
    b0all = b0all_ref[...]
    outs = (ef0_ref, ef1_ref, ef2_ref, ef3_ref)
    for l in range(4):
        outs[l][...] = (jnp.dot(ef, weall[:, l * 64:(l + 1) * 64],
                                preferred_element_type=jnp.float32)
                        + b0all[:, l * 64:(l + 1) * 64])


def _k2_call(ev, wd0, bd0, wd1, bd1, wr0p, br0, wr1, br1, g, b, weall, b0all):
    grid = (N_EDGES // EB_ROWS,)
    full = lambda shp: pl.BlockSpec(shp, lambda i: (0,) * len(shp))
    return pl.pallas_call(
        _k2_body,
        grid=grid,
        in_specs=[
            pl.BlockSpec((EB_ROWS, 16), lambda i: (i, 0)),
            full((1, 32)), full((1, 32)),
            full((32, 32)), full((1, 32)),
            full((8, 32)), full((1, 32)),
            full((32, 32)), full((1, 32)),
            full((1, HIDK)), full((1, HIDK)),
            full((HIDK, 256)), full((1, 256)),
        ],
        out_specs=[pl.BlockSpec((EB_ROWS, HIDK), lambda i: (i, 0))] * 4,
        out_shape=[jax.ShapeDtypeStruct((N_EDGES, HIDK), jnp.float32)] * 4,
    )(ev, wd0, bd0, wd1, bd1, wr0p, br0, wr1, br1, g, b, weall, b0all)


def _sc_layer_body(xr_hbm, xc_hbm, row_hbm, col_hbm, ef_hbm, out_hbm,
                   xr_sh, xc_sh, h_sh, idx_r, idx_c, efb, gr, gc, hb, stage,
                   sem_a, sem_b):
    c = lax.axis_index("c")
    s = lax.axis_index("s")
    wid = c * 16 + s
    r0 = s * ROWS_PER_TILE

    def zbody(i, _):
        for j in range(4):
            stage[i, pl.ds(j * 16, 16)] = jnp.zeros((16,), jnp.float32)
        return 0
    lax.fori_loop(0, ROWS_PER_TILE, zbody, 0)
    pltpu.sync_copy(stage, h_sh.at[pl.ds(r0, ROWS_PER_TILE)])
    pltpu.sync_copy(xr_hbm.at[pl.ds(r0, ROWS_PER_TILE)], stage)
    pltpu.sync_copy(stage, xr_sh.at[pl.ds(r0, ROWS_PER_TILE)])
    pltpu.sync_copy(xc_hbm.at[pl.ds(r0, ROWS_PER_TILE)], stage)
    pltpu.sync_copy(stage, xc_sh.at[pl.ds(r0, ROWS_PER_TILE)])
    plsc.subcore_barrier()

    base = wid * EDGES_PER_TILE

    def chunk(k, _):
        e0 = base + k * SC_CHUNK
        pltpu.sync_copy(row_hbm.at[pl.ds(e0, SC_CHUNK)], idx_r)
        pltpu.sync_copy(col_hbm.at[pl.ds(e0, SC_CHUNK)], idx_c)
        pltpu.sync_copy(ef_hbm.at[pl.ds(e0, SC_CHUNK)], efb)
        cp_r = pltpu.async_copy(xr_sh.at[idx_r], gr, sem_a)
        cp_c = pltpu.async_copy(xc_sh.at[idx_c], gc, sem_b)
        cp_r.wait()
        cp_c.wait()

        def ebody(i, _):
            for j in range(4):
                sl = pl.ds(j * 16, 16)
                t = gr[i, sl] + gc[i, sl] + efb[i, sl]
                sg = 1.0 / (1.0 + jnp.exp(-t))
                hb[i, sl] = t * sg
            return 0
        lax.fori_loop(0, SC_CHUNK, ebody, 0)
        pltpu.sync_copy(hb, h_sh.at[idx_r], add=True)
        return 0

    lax.fori_loop(0, N_CHUNKS, chunk, 0)
    plsc.subcore_barrier()
    pltpu.sync_copy(h_sh.at[pl.ds(r0, ROWS_PER_TILE)], stage)
    pltpu.sync_copy(stage, out_hbm.at[c, pl.ds(r0, ROWS_PER_TILE)])


def _sc_layer_call(xr, xc, row, col, ef):
    mesh = plsc.VectorSubcoreMesh(core_axis_name="c", subcore_axis_name="s")
    kfn = functools.partial(
        pl.kernel,
        mesh=mesh,
        out_type=jax.ShapeDtypeStruct((2, N_NODES, HIDK), jnp.float32),
        scratch_types=[
            pltpu.VMEM_SHARED((N_NODES, HIDK), jnp.float32),
            pltpu.VMEM_SHARED((N_NODES, HIDK), jnp.float32),
            pltpu.VMEM_SHARED((N_NODES, HIDK), jnp.float32),
            pltpu.VMEM((SC_CHUNK,), jnp.int32),
            pltpu.VMEM((SC_CHUNK,), jnp.int32),
            pltpu.VMEM((SC_CHUNK, HIDK), jnp.float32),
            pltpu.VMEM((SC_CHUNK, HIDK), jnp.float32),
            pltpu.VMEM((SC_CHUNK, HIDK), jnp.float32),
            pltpu.VMEM((SC_CHUNK, HIDK), jnp.float32),
            pltpu.VMEM((ROWS_PER_TILE, HIDK), jnp.float32),
            pltpu.SemaphoreType.DMA,
            pltpu.SemaphoreType.DMA,
        ],
    )
    return kfn(_sc_layer_body)(xr, xc, row, col, ef)


def _k3_body(with_next, x_ref, hp0_ref, hp1_ref, dg0_ref, dg1_ref,
             w1_ref, b1_ref, g1_ref, bl1_ref,
             wf0_ref, bf0_ref, wf1_ref, bf1_ref, g2_ref, bl2_ref,
             wrn_ref, wcn_ref,
             x_out, xr_out=None, xc_out=None):
    x = x_ref[...]
    hs = hp0_ref[...] + hp1_ref[...]
    deg = dg0_ref[...][:, 0:1] + dg1_ref[...][:, 0:1]
    m = jnp.dot(hs, w1_ref[...], preferred_element_type=jnp.float32) + b1_ref[...] * deg
    x = _ln2(x + m, g1_ref[...], bl1_ref[...])
    f = _silu(jnp.dot(x, wf0_ref[...], preferred_element_type=jnp.float32) + bf0_ref[...])
    f = jnp.dot(f, wf1_ref[...], preferred_element_type=jnp.float32) + bf1_ref[...]
    x = _ln2(x + f, g2_ref[...], bl2_ref[...])
    x_out[...] = x
    if with_next:
        xr_out[...] = jnp.dot(x, wrn_ref[...], preferred_element_type=jnp.float32)
        xc_out[...] = jnp.dot(x, wcn_ref[...], preferred_element_type=jnp.float32)


def _k3_call(with_next, x, hp0, hp1, dg0, dg1,
             w1, b1, g1, bl1, wf0, bf0, wf1, bf1, g2, bl2, wrn, wcn):
    n = x.shape[0]
    grid = (n // NB_ROWS,)
    full = lambda shp: pl.BlockSpec(shp, lambda i: (0,) * len(shp))
    rowspec = pl.BlockSpec((NB_ROWS, HIDK), lambda i: (i, 0))
    n_out = 3 if with_next else 1
    outs = pl.pallas_call(
        functools.partial(_k3_body, with_next),
        grid=grid,
        in_specs=[
            rowspec, rowspec, rowspec,
            pl.BlockSpec((NB_ROWS, 16), lambda i: (i, 0)),
            pl.BlockSpec((NB_ROWS, 16), lambda i: (i, 0)),
            full((HIDK, HIDK)), full((1, HIDK)), full((1, HIDK)), full((1, HIDK)),
            full((HIDK, 2 * HIDK)), full((1, 2 * HIDK)),
            full((2 * HIDK, HIDK)), full((1, HIDK)),
            full((1, HIDK)), full((1, HIDK)),
            full((HIDK, HIDK)), full((HIDK, HIDK)),
        ],
        out_specs=[rowspec] * n_out,
        out_shape=[jax.ShapeDtypeStruct((n, HIDK), jnp.float32)] * n_out,
    )(x, hp0, hp1, dg0, dg1, w1, b1, g1, bl1, wf0, bf0, wf1, bf1, g2, bl2, wrn, wcn)
    if with_next:
        return outs
    return outs[0], None, None


def _k4_body(x_ref, batch_ref, wpp_ref, bpp_ref,
             wm0_ref, bm0_ref, wm1_ref, bm1_ref,
             ws0_ref, bs0_ref, ws1_ref, bs1_ref,
             macc_ref, mxacc_ref, cacc_ref, mu_ref, sig_ref):
    i = pl.program_id(0)
    nsteps = pl.num_programs(0)

    @pl.when(i == 0)
    def _init():
        macc_ref[...] = jnp.zeros((32, HIDK), jnp.float32)
        mxacc_ref[...] = jnp.full((32, HIDK), -1e30, jnp.float32)
        cacc_ref[...] = jnp.zeros((32, HIDK), jnp.float32)

    xb = x_ref[...]
    bb = batch_ref[...]
    oh = (bb == lax.broadcasted_iota(jnp.int32, (1, 32), 1)).astype(jnp.float32)
    macc_ref[...] += lax.dot_general(oh, xb, (((0,), (0,)), ((), ())),
                                     preferred_element_type=jnp.float32)
    cacc_ref[...] += jnp.broadcast_to(jnp.sum(oh, axis=0)[:, None], (32, HIDK))
    rows = []
    for bnum in range(32):
        msk = (bb == bnum)
        rowmax = jnp.max(jnp.where(msk, xb, -1e30), axis=0)
        rows.append(rowmax[None, :])
    mxacc_ref[...] = jnp.maximum(mxacc_ref[...], jnp.concatenate(rows, axis=0))

    @pl.when(i == nsteps - 1)
    def _final():
        counts = jnp.maximum(cacc_ref[...], 1.0)
        mean = macc_ref[...] / counts
        pooled = jnp.concatenate([mean, mxacc_ref[...]], axis=-1)
        pooled = jnp.dot(pooled, wpp_ref[...], preferred_element_type=jnp.float32) + bpp_ref[...]
        hm = _silu(jnp.dot(pooled, wm0_ref[...], preferred_element_type=jnp.float32) + bm0_ref[...])
        mu_ref[...] = jax.nn.sigmoid(
            jnp.dot(hm, wm1_ref[...], preferred_element_type=jnp.float32) + bm1_ref[...])
        hsg = _silu(jnp.dot(pooled, ws0_ref[...], preferred_element_type=jnp.float32) + bs0_ref[...])
        sig_ref[...] = jax.nn.softplus(
            jnp.dot(hsg, ws1_ref[...], preferred_element_type=jnp.float32) + bs1_ref[...])


def _k4_call(x, batch2, wpp, bpp, wm0, bm0, wm1p, bm1p, ws0, bs0, ws1p, bs1p):
    n = x.shape[0]
    grid = (n // NB_ROWS,)
    full = lambda shp: pl.BlockSpec(shp, lambda i: (0,) * len(shp))
    return pl.pallas_call(
        _k4_body,
        grid=grid,
        in_specs=[
            pl.BlockSpec((NB_ROWS, HIDK), lambda i: (i, 0)),
            pl.BlockSpec((NB_ROWS, 1), lambda i: (i, 0)),
            full((2 * HIDK, HIDK)), full((1, HIDK)),
            full((HIDK, HIDK)), full((1, HIDK)),
            full((HIDK, HIDK)), full((1, HIDK)),
            full((HIDK, HIDK)), full((1, HIDK)),
            full((HIDK, HIDK)), full((1, HIDK)),
        ],
        out_specs=[full((32, HIDK))] * 5,
        out_shape=[jax.ShapeDtypeStruct((32, HIDK), jnp.float32)] * 5,
    )(x, batch2, wpp, bpp, wm0, bm0, wm1p, bm1p, ws0, bs0, ws1p, bs1p)


def kernel(theta_t, pos_t, t, batch, edge_index, params):
    n = theta_t.shape[0]

    r1 = lambda v: v.reshape(1, -1)

    theta_pad = jnp.pad(theta_t, ((0, 0), (0, 128 - theta_t.shape[1])),
                        constant_values=-1e30)
    wn_pad = jnp.pad(params["node_in"]["W"], ((0, 128 - theta_t.shape[1]), (0, 0)))
    batch2 = batch.astype(jnp.int32).reshape(n, 1)
    row = edge_index[0].astype(jnp.int32)
    col = edge_index[1].astype(jnp.int32)
    pos_pad = jnp.pad(pos_t, ((0, 0), (0, 13)))

    half = TDIMK // 2
    inv_freq = 1.0 / (10000.0 ** (jnp.arange(half, dtype=jnp.float32) / half))
    sin_inp = t[:, None] * inv_freq[None, :]
    emb = jnp.concatenate([jnp.sin(sin_inp), jnp.cos(sin_inp)], axis=-1)

    blocks = params["blocks"]
    wr_l = [b["msg0"]["W"][:HIDK] for b in blocks]
    wc_l = [b["msg0"]["W"][HIDK:2 * HIDK] for b in blocks]
    weall = jnp.concatenate([b["msg0"]["W"][2 * HIDK:] for b in blocks], axis=1)
    b0all = jnp.concatenate([b["msg0"]["b"] for b in blocks]).reshape(1, 256)

    x, xr, xc = _k1_call(
        theta_pad, batch2, emb, wn_pad, r1(params["node_in"]["b"]),
        params["tproj0"]["W"], r1(params["tproj0"]["b"]),
        params["tproj1"]["W"], r1(params["tproj1"]["b"]),
        params["time_proj"]["W"], r1(params["time_proj"]["b"]),
        wr_l[0], wc_l[0])

    ev, degp = _pre_call(pos_pad, row, col)
    dg0 = degp[0]
    dg1 = degp[1]

    wr0p = jnp.pad(params["dir0"]["W"], ((0, 5), (0, 0)))
    efs = _k2_call(
        ev, r1(params["dist0"]["W"][0]), r1(params["dist0"]["b"]),
        params["dist1"]["W"], r1(params["dist1"]["b"]),
        wr0p, r1(params["dir0"]["b"]),
        params["dir1"]["W"], r1(params["dir1"]["b"]),
        r1(params["edge_norm"]["g"]), r1(params["edge_norm"]["b"]),
        weall, b0all)

    for l, blk in enumerate(blocks):
        hp = _sc_layer_call(xr, xc, row, col, efs[l])
        with_next = l < len(blocks) - 1
        wrn = wr_l[l + 1] if with_next else wr_l[0]
        wcn = wc_l[l + 1] if with_next else wc_l[0]
        x, xr, xc = _k3_call(
            with_next, x, hp[0], hp[1], dg0, dg1,
            blk["msg1"]["W"], r1(blk["msg1"]["b"]),
            r1(blk["ln1"]["g"]), r1(blk["ln1"]["b"]),
            blk["ffn0"]["W"], r1(blk["ffn0"]["b"]),
            blk["ffn1"]["W"], r1(blk["ffn1"]["b"]),
            r1(blk["ln2"]["g"]), r1(blk["ln2"]["b"]),
            wrn, wcn)

    wm1p = jnp.pad(params["mu1"]["W"], ((0, 0), (0, HIDK - 2)))
    bm1p = jnp.pad(r1(params["mu1"]["b"]), ((0, 0), (0, HIDK - 2)))
    ws1p = jnp.pad(params["sig1"]["W"], ((0, 0), (0, HIDK - 2)))
    bs1p = jnp.pad(r1(params["sig1"]["b"]), ((0, 0), (0, HIDK - 2)))
    _, _, _, mu, sig = _k4_call(
        x, batch2, params["pool_proj"]["W"], r1(params["pool_proj"]["b"]),
        params["mu0"]["W"], r1(params["mu0"]["b"]), wm1p, bm1p,
        params["sig0"]["W"], r1(params["sig0"]["b"]), ws1p, bs1p)
    return mu[:, :2], sig[:, :2]


# speedup vs baseline: 3.2156x; 3.2156x over previous
import functools
import jax
import jax.numpy as jnp
from jax import lax
from jax.experimental import pallas as pl
from jax.experimental.pallas import tpu as pltpu
from jax.experimental.pallas import tpu_sc as plsc

HIDK = 64
TDIMK = 64

N_NODES = 10240
N_EDGES = 320000
NB_ROWS = 1024
EB_ROWS = 512
SC_CHUNK = 80
ROWS_PER_TILE = N_NODES // 16
EDGES_PER_TILE = N_EDGES // 32
N_CHUNKS = EDGES_PER_TILE // SC_CHUNK


def _silu(x):
    return x * jax.nn.sigmoid(x)


def _ln2(x, g, b):
    mu = jnp.mean(x, axis=-1, keepdims=True)
    var = jnp.var(x, axis=-1, keepdims=True)
    return (x - mu) / jnp.sqrt(var + 1e-5) * g + b


def _k1_body(theta_ref, batch_ref, emb_ref, wn_ref, bn_ref,
             wt0_ref, bt0_ref, wt1_ref, bt1_ref, wtp_ref, btp_ref,
             wr_ref, wc_ref,
             x_ref, xr_ref, xc_ref):
    th = theta_ref[...]
    m = jnp.max(th, axis=-1, keepdims=True)
    e = jnp.exp(th - m)
    sm = e / jnp.sum(e, axis=-1, keepdims=True)
    x = jnp.dot(sm, wn_ref[...], preferred_element_type=jnp.float32) + bn_ref[...]
    emb = emb_ref[...]
    t0 = _silu(jnp.dot(emb, wt0_ref[...], preferred_element_type=jnp.float32) + bt0_ref[...])
    temb = jnp.dot(t0, wt1_ref[...], preferred_element_type=jnp.float32) + bt1_ref[...]
    temb = jnp.dot(temb, wtp_ref[...], preferred_element_type=jnp.float32) + btp_ref[...]
    bb = batch_ref[...]
    oh = (bb == lax.broadcasted_iota(jnp.int32, (1, 32), 1)).astype(jnp.float32)
    x = x + jnp.dot(oh, temb, preferred_element_type=jnp.float32)
    x_ref[...] = x
    xr_ref[...] = jnp.dot(x, wr_ref[...], preferred_element_type=jnp.float32)
    xc_ref[...] = jnp.dot(x, wc_ref[...], preferred_element_type=jnp.float32)


def _k1_call(theta_pad, batch2, emb, wn_pad, bn, wt0, bt0, wt1, bt1, wtp, btp, wr, wc):
    n = theta_pad.shape[0]
    grid = (n // NB_ROWS,)
    full = lambda shp: pl.BlockSpec(shp, lambda i: (0,) * len(shp))
    return pl.pallas_call(
        _k1_body,
        grid=grid,
        in_specs=[
            pl.BlockSpec((NB_ROWS, 128), lambda i: (i, 0)),
            pl.BlockSpec((NB_ROWS, 1), lambda i: (i, 0)),
            full((32, TDIMK)),
            full((128, HIDK)), full((1, HIDK)),
            full((TDIMK, TDIMK)), full((1, TDIMK)),
            full((TDIMK, TDIMK)), full((1, TDIMK)),
            full((TDIMK, HIDK)), full((1, HIDK)),
            full((HIDK, HIDK)), full((HIDK, HIDK)),
        ],
        out_specs=[pl.BlockSpec((NB_ROWS, HIDK), lambda i: (i, 0))] * 3,
        out_shape=[jax.ShapeDtypeStruct((n, HIDK), jnp.float32)] * 3,
    )(theta_pad, batch2, emb, wn_pad, bn, wt0, bt0, wt1, bt1, wtp, btp, wr, wc)


def _pre_kernel_body(pos_hbm, row_hbm, col_hbm, ev_out, deg_out,
                     pos_sh, deg_sh, idx_r, idx_c, pr, pc, evb, onesb, stage,
                     sem_a, sem_b):
    c = lax.axis_index("c")
    s = lax.axis_index("s")
    wid = c * 16 + s
    r0 = s * ROWS_PER_TILE

    def zbody(i, _):
        stage[i, :] = jnp.zeros((16,), jnp.float32)
        return 0
    lax.fori_loop(0, ROWS_PER_TILE, zbody, 0)
    pltpu.sync_copy(stage, deg_sh.at[pl.ds(r0, ROWS_PER_TILE)])
    pltpu.sync_copy(pos_hbm.at[pl.ds(r0, ROWS_PER_TILE)], stage)
    pltpu.sync_copy(stage, pos_sh.at[pl.ds(r0, ROWS_PER_TILE)])

    def obody(i, _):
        onesb[i, :] = jnp.full((16,), 1.0, jnp.float32)
        return 0
    lax.fori_loop(0, SC_CHUNK, obody, 0)
    plsc.subcore_barrier()

    base = wid * EDGES_PER_TILE

    def chunk(k, _):
        e0 = base + k * SC_CHUNK
        pltpu.sync_copy(row_hbm.at[pl.ds(e0, SC_CHUNK)], idx_r)
        pltpu.sync_copy(col_hbm.at[pl.ds(e0, SC_CHUNK)], idx_c)
        cp_r = pltpu.async_copy(pos_sh.at[idx_r], pr, sem_a)
        cp_c = pltpu.async_copy(pos_sh.at[idx_c], pc, sem_b)
        cp_r.wait()
        cp_c.wait()

        def ebody(i, _):
            evb[i, :] = pc[i, :] - pr[i, :]
            return 0
        lax.fori_loop(0, SC_CHUNK, ebody, 0)
        pltpu.sync_copy(evb, ev_out.at[pl.ds(e0, SC_CHUNK)])
        pltpu.sync_copy(onesb, deg_sh.at[idx_r], add=True)
        return 0

    lax.fori_loop(0, N_CHUNKS, chunk, 0)
    plsc.subcore_barrier()
    pltpu.sync_copy(deg_sh.at[pl.ds(r0, ROWS_PER_TILE)], stage)
    pltpu.sync_copy(stage, deg_out.at[c, pl.ds(r0, ROWS_PER_TILE)])


def _pre_call(pos_pad, row, col):
    mesh = plsc.VectorSubcoreMesh(core_axis_name="c", subcore_axis_name="s")
    kfn = functools.partial(
        pl.kernel,
        mesh=mesh,
        compiler_params=pltpu.CompilerParams(use_tc_tiling_on_sc=False),
        out_type=[
            jax.ShapeDtypeStruct((N_EDGES, 16), jnp.float32),
            jax.ShapeDtypeStruct((2, N_NODES, 16), jnp.float32),
        ],
        scratch_types=[
            pltpu.VMEM_SHARED((N_NODES, 16), jnp.float32),
            pltpu.VMEM_SHARED((N_NODES, 16), jnp.float32),
            pltpu.VMEM((SC_CHUNK,), jnp.int32),
            pltpu.VMEM((SC_CHUNK,), jnp.int32),
            pltpu.VMEM((SC_CHUNK, 16), jnp.float32),
            pltpu.VMEM((SC_CHUNK, 16), jnp.float32),
            pltpu.VMEM((SC_CHUNK, 16), jnp.float32),
            pltpu.VMEM((SC_CHUNK, 16), jnp.float32),
            pltpu.VMEM((ROWS_PER_TILE, 16), jnp.float32),
            pltpu.SemaphoreType.DMA,
            pltpu.SemaphoreType.DMA,
        ],
    )
    return kfn(_pre_kernel_body)(pos_pad, row, col)


def _k2_body(ev_ref, wd0_ref, bd0_ref, wd1_ref, bd1_ref,
             wr0_ref, br0_ref, wr1_ref, br1_ref,
             g_ref, b_ref, weall_ref, b0all_ref,
             ef0_ref, ef1_ref, ef2_ref, ef3_ref):
    ev = ev_ref[...][:, 0:3]
    d2 = jnp.sum(ev * ev, axis=-1, keepdims=True)
    d = jnp.sqrt(d2)
    dirv = ev / (d + 1e-8)
    h = _silu(d * wd0_ref[...] + bd0_ref[...])
    df = jnp.dot(h, wd1_ref[...], preferred_element_type=jnp.float32) + bd1_ref[...]
    wr0 = wr0_ref[...]
    rh = _silu(dirv[:, 0:1] * wr0[0:1, :] + dirv[:, 1:2] * wr0[1:2, :]
               + dirv[:, 2:3] * wr0[2:3, :] + br0_ref[...])
    rf = jnp.dot(rh, wr1_ref[...], preferred_element_type=jnp.float32) + br1_ref[...]
    ef = jnp.concatenate([df, rf], axis=-1)
    ef = _ln2(ef, g_ref[...], b_ref[...])
    weall = weall_ref[...]
    b0all = b0all_ref[...]
    outs = (ef0_ref, ef1_ref, ef2_ref, ef3_ref)
    for l in range(4):
        outs[l][...] = (jnp.dot(ef, weall[:, l * 64:(l + 1) * 64],
                                preferred_element_type=jnp.float32)
                        + b0all[:, l * 64:(l + 1) * 64])


def _k2_call(ev, wd0, bd0, wd1, bd1, wr0p, br0, wr1, br1, g, b, weall, b0all):
    grid = (N_EDGES // EB_ROWS,)
    full = lambda shp: pl.BlockSpec(shp, lambda i: (0,) * len(shp))
    return pl.pallas_call(
        _k2_body,
        grid=grid,
        in_specs=[
            pl.BlockSpec((EB_ROWS, 16), lambda i: (i, 0)),
            full((1, 32)), full((1, 32)),
            full((32, 32)), full((1, 32)),
            full((8, 32)), full((1, 32)),
            full((32, 32)), full((1, 32)),
            full((1, HIDK)), full((1, HIDK)),
            full((HIDK, 256)), full((1, 256)),
        ],
        out_specs=[pl.BlockSpec((EB_ROWS, HIDK), lambda i: (i, 0))] * 4,
        out_shape=[jax.ShapeDtypeStruct((N_EDGES, HIDK), jnp.float32)] * 4,
    )(ev, wd0, bd0, wd1, bd1, wr0p, br0, wr1, br1, g, b, weall, b0all)


def _sc_layer_body(xr_hbm, xc_hbm, row_hbm, col_hbm, ef_hbm, out_hbm,
                   xr_sh, h_sh, idx_r, idx_c, efb, gr, gc, hb,
                   sem_a, sem_b):
    c = lax.axis_index("c")
    s = lax.axis_index("s")
    wid = c * 16 + s
    r0 = s * ROWS_PER_TILE
    n_stage = ROWS_PER_TILE // SC_CHUNK

    def zbody(i, _):
        for j in range(4):
            efb[i, pl.ds(j * 16, 16)] = jnp.zeros((16,), jnp.float32)
        return 0
    lax.fori_loop(0, SC_CHUNK, zbody, 0)
    for q in range(n_stage):
        pltpu.sync_copy(efb, h_sh.at[pl.ds(r0 + q * SC_CHUNK, SC_CHUNK)])
    for q in range(n_stage):
        pltpu.sync_copy(xr_hbm.at[pl.ds(r0 + q * SC_CHUNK, SC_CHUNK)], gr)
        pltpu.sync_copy(gr, xr_sh.at[pl.ds(r0 + q * SC_CHUNK, SC_CHUNK)])
    plsc.subcore_barrier()

    base = wid * EDGES_PER_TILE

    def chunk(k, _):
        e0 = base + k * SC_CHUNK
        pltpu.sync_copy(row_hbm.at[pl.ds(e0, SC_CHUNK)], idx_r)
        pltpu.sync_copy(col_hbm.at[pl.ds(e0, SC_CHUNK)], idx_c)
        pltpu.sync_copy(ef_hbm.at[pl.ds(e0, SC_CHUNK)], efb)
        cp_c = pltpu.async_copy(xc_hbm.at[idx_c], gc, sem_b)
        cp_r = pltpu.async_copy(xr_sh.at[idx_r], gr, sem_a)
        cp_r.wait()
        cp_c.wait()

        def ebody(i, _):
            for j in range(4):
                sl = pl.ds(j * 16, 16)
                t = gr[i, sl] + gc[i, sl] + efb[i, sl]
                sg = 1.0 / (1.0 + jnp.exp(-t))
                hb[i, sl] = t * sg
            return 0
        lax.fori_loop(0, SC_CHUNK, ebody, 0)
        pltpu.sync_copy(hb, h_sh.at[idx_r], add=True)
        return 0

    lax.fori_loop(0, N_CHUNKS, chunk, 0)
    plsc.subcore_barrier()
    for q in range(n_stage):
        pltpu.sync_copy(h_sh.at[pl.ds(r0 + q * SC_CHUNK, SC_CHUNK)], hb)
        pltpu.sync_copy(hb, out_hbm.at[c, pl.ds(r0 + q * SC_CHUNK, SC_CHUNK)])


def _sc_layer_call(xr, xc, row, col, ef):
    mesh = plsc.VectorSubcoreMesh(core_axis_name="c", subcore_axis_name="s")
    kfn = functools.partial(
        pl.kernel,
        mesh=mesh,
        compiler_params=pltpu.CompilerParams(use_tc_tiling_on_sc=False),
        out_type=jax.ShapeDtypeStruct((2, N_NODES, HIDK), jnp.float32),
        scratch_types=[
            pltpu.VMEM_SHARED((N_NODES, HIDK), jnp.float32),
            pltpu.VMEM_SHARED((N_NODES, HIDK), jnp.float32),
            pltpu.VMEM((SC_CHUNK,), jnp.int32),
            pltpu.VMEM((SC_CHUNK,), jnp.int32),
            pltpu.VMEM((SC_CHUNK, HIDK), jnp.float32),
            pltpu.VMEM((SC_CHUNK, HIDK), jnp.float32),
            pltpu.VMEM((SC_CHUNK, HIDK), jnp.float32),
            pltpu.VMEM((SC_CHUNK, HIDK), jnp.float32),
            pltpu.SemaphoreType.DMA,
            pltpu.SemaphoreType.DMA,
        ],
    )
    return kfn(_sc_layer_body)(xr, xc, row, col, ef)


def _k3_body(with_next, x_ref, hp0_ref, hp1_ref, dg0_ref, dg1_ref,
             w1_ref, b1_ref, g1_ref, bl1_ref,
             wf0_ref, bf0_ref, wf1_ref, bf1_ref, g2_ref, bl2_ref,
             wrn_ref, wcn_ref,
             x_out, xr_out=None, xc_out=None):
    x = x_ref[...]
    hs = hp0_ref[...] + hp1_ref[...]
    deg = dg0_ref[...][:, 0:1] + dg1_ref[...][:, 0:1]
    m = jnp.dot(hs, w1_ref[...], preferred_element_type=jnp.float32) + b1_ref[...] * deg
    x = _ln2(x + m, g1_ref[...], bl1_ref[...])
    f = _silu(jnp.dot(x, wf0_ref[...], preferred_element_type=jnp.float32) + bf0_ref[...])
    f = jnp.dot(f, wf1_ref[...], preferred_element_type=jnp.float32) + bf1_ref[...]
    x = _ln2(x + f, g2_ref[...], bl2_ref[...])
    x_out[...] = x
    if with_next:
        xr_out[...] = jnp.dot(x, wrn_ref[...], preferred_element_type=jnp.float32)
        xc_out[...] = jnp.dot(x, wcn_ref[...], preferred_element_type=jnp.float32)


def _k3_call(with_next, x, hp0, hp1, dg0, dg1,
             w1, b1, g1, bl1, wf0, bf0, wf1, bf1, g2, bl2, wrn, wcn):
    n = x.shape[0]
    grid = (n // NB_ROWS,)
    full = lambda shp: pl.BlockSpec(shp, lambda i: (0,) * len(shp))
    rowspec = pl.BlockSpec((NB_ROWS, HIDK), lambda i: (i, 0))
    n_out = 3 if with_next else 1
    outs = pl.pallas_call(
        functools.partial(_k3_body, with_next),
        grid=grid,
        in_specs=[
            rowspec, rowspec, rowspec,
            pl.BlockSpec((NB_ROWS, 16), lambda i: (i, 0)),
            pl.BlockSpec((NB_ROWS, 16), lambda i: (i, 0)),
            full((HIDK, HIDK)), full((1, HIDK)), full((1, HIDK)), full((1, HIDK)),
            full((HIDK, 2 * HIDK)), full((1, 2 * HIDK)),
            full((2 * HIDK, HIDK)), full((1, HIDK)),
            full((1, HIDK)), full((1, HIDK)),
            full((HIDK, HIDK)), full((HIDK, HIDK)),
        ],
        out_specs=[rowspec] * n_out,
        out_shape=[jax.ShapeDtypeStruct((n, HIDK), jnp.float32)] * n_out,
    )(x, hp0, hp1, dg0, dg1, w1, b1, g1, bl1, wf0, bf0, wf1, bf1, g2, bl2, wrn, wcn)
    if with_next:
        return outs
    return outs[0], None, None


def _k4_body(x_ref, batch_ref, wpp_ref, bpp_ref,
             wm0_ref, bm0_ref, wm1_ref, bm1_ref,
             ws0_ref, bs0_ref, ws1_ref, bs1_ref,
             macc_ref, mxacc_ref, cacc_ref, mu_ref, sig_ref):
    i = pl.program_id(0)
    nsteps = pl.num_programs(0)

    @pl.when(i == 0)
    def _init():
        macc_ref[...] = jnp.zeros((32, HIDK), jnp.float32)
        mxacc_ref[...] = jnp.full((32, HIDK), -1e30, jnp.float32)
        cacc_ref[...] = jnp.zeros((32, HIDK), jnp.float32)

    xb = x_ref[...]
    bb = batch_ref[...]
    oh = (bb == lax.broadcasted_iota(jnp.int32, (1, 32), 1)).astype(jnp.float32)
    macc_ref[...] += lax.dot_general(oh, xb, (((0,), (0,)), ((), ())),
                                     preferred_element_type=jnp.float32)
    cacc_ref[...] += jnp.broadcast_to(jnp.sum(oh, axis=0)[:, None], (32, HIDK))
    rows = []
    for bnum in range(32):
        msk = (bb == bnum)
        rowmax = jnp.max(jnp.where(msk, xb, -1e30), axis=0)
        rows.append(rowmax[None, :])
    mxacc_ref[...] = jnp.maximum(mxacc_ref[...], jnp.concatenate(rows, axis=0))

    @pl.when(i == nsteps - 1)
    def _final():
        counts = jnp.maximum(cacc_ref[...], 1.0)
        mean = macc_ref[...] / counts
        pooled = jnp.concatenate([mean, mxacc_ref[...]], axis=-1)
        pooled = jnp.dot(pooled, wpp_ref[...], preferred_element_type=jnp.float32) + bpp_ref[...]
        hm = _silu(jnp.dot(pooled, wm0_ref[...], preferred_element_type=jnp.float32) + bm0_ref[...])
        mu_ref[...] = jax.nn.sigmoid(
            jnp.dot(hm, wm1_ref[...], preferred_element_type=jnp.float32) + bm1_ref[...])
        hsg = _silu(jnp.dot(pooled, ws0_ref[...], preferred_element_type=jnp.float32) + bs0_ref[...])
        sig_ref[...] = jax.nn.softplus(
            jnp.dot(hsg, ws1_ref[...], preferred_element_type=jnp.float32) + bs1_ref[...])


def _k4_call(x, batch2, wpp, bpp, wm0, bm0, wm1p, bm1p, ws0, bs0, ws1p, bs1p):
    n = x.shape[0]
    grid = (n // NB_ROWS,)
    full = lambda shp: pl.BlockSpec(shp, lambda i: (0,) * len(shp))
    return pl.pallas_call(
        _k4_body,
        grid=grid,
        in_specs=[
            pl.BlockSpec((NB_ROWS, HIDK), lambda i: (i, 0)),
            pl.BlockSpec((NB_ROWS, 1), lambda i: (i, 0)),
            full((2 * HIDK, HIDK)), full((1, HIDK)),
            full((HIDK, HIDK)), full((1, HIDK)),
            full((HIDK, HIDK)), full((1, HIDK)),
            full((HIDK, HIDK)), full((1, HIDK)),
            full((HIDK, HIDK)), full((1, HIDK)),
        ],
        out_specs=[full((32, HIDK))] * 5,
        out_shape=[jax.ShapeDtypeStruct((32, HIDK), jnp.float32)] * 5,
    )(x, batch2, wpp, bpp, wm0, bm0, wm1p, bm1p, ws0, bs0, ws1p, bs1p)


def kernel(theta_t, pos_t, t, batch, edge_index, params):
    n = theta_t.shape[0]
    npad = N_NODES - n

    r1 = lambda v: v.reshape(1, -1)

    theta_pad = jnp.pad(theta_t, ((0, npad), (0, 128 - theta_t.shape[1])),
                        constant_values=-1e30)
    wn_pad = jnp.pad(params["node_in"]["W"], ((0, 128 - theta_t.shape[1]), (0, 0)))
    batch2 = jnp.pad(batch.astype(jnp.int32), (0, npad),
                     constant_values=99).reshape(N_NODES, 1)
    row = edge_index[0].astype(jnp.int32)
    col = edge_index[1].astype(jnp.int32)
    pos_pad = jnp.pad(pos_t, ((0, npad), (0, 13)))

    half = TDIMK // 2
    inv_freq = 1.0 / (10000.0 ** (jnp.arange(half, dtype=jnp.float32) / half))
    sin_inp = t[:, None] * inv_freq[None, :]
    emb = jnp.concatenate([jnp.sin(sin_inp), jnp.cos(sin_inp)], axis=-1)

    blocks = params["blocks"]
    wr_l = [b["msg0"]["W"][:HIDK] for b in blocks]
    wc_l = [b["msg0"]["W"][HIDK:2 * HIDK] for b in blocks]
    weall = jnp.concatenate([b["msg0"]["W"][2 * HIDK:] for b in blocks], axis=1)
    b0all = jnp.concatenate([b["msg0"]["b"] for b in blocks]).reshape(1, 256)

    x, xr, xc = _k1_call(
        theta_pad, batch2, emb, wn_pad, r1(params["node_in"]["b"]),
        params["tproj0"]["W"], r1(params["tproj0"]["b"]),
        params["tproj1"]["W"], r1(params["tproj1"]["b"]),
        params["time_proj"]["W"], r1(params["time_proj"]["b"]),
        wr_l[0], wc_l[0])

    ev, degp = _pre_call(pos_pad, row, col)
    dg0 = degp[0]
    dg1 = degp[1]

    wr0p = jnp.pad(params["dir0"]["W"], ((0, 5), (0, 0)))
    efs = _k2_call(
        ev, r1(params["dist0"]["W"][0]), r1(params["dist0"]["b"]),
        params["dist1"]["W"], r1(params["dist1"]["b"]),
        wr0p, r1(params["dir0"]["b"]),
        params["dir1"]["W"], r1(params["dir1"]["b"]),
        r1(params["edge_norm"]["g"]), r1(params["edge_norm"]["b"]),
        weall, b0all)

    for l, blk in enumerate(blocks):
        hp = _sc_layer_call(xr, xc, row, col, efs[l])
        with_next = l < len(blocks) - 1
        wrn = wr_l[l + 1] if with_next else wr_l[0]
        wcn = wc_l[l + 1] if with_next else wc_l[0]
        x, xr, xc = _k3_call(
            with_next, x, hp[0], hp[1], dg0, dg1,
            blk["msg1"]["W"], r1(blk["msg1"]["b"]),
            r1(blk["ln1"]["g"]), r1(blk["ln1"]["b"]),
            blk["ffn0"]["W"], r1(blk["ffn0"]["b"]),
            blk["ffn1"]["W"], r1(blk["ffn1"]["b"]),
            r1(blk["ln2"]["g"]), r1(blk["ln2"]["b"]),
            wrn, wcn)

    wm1p = jnp.pad(params["mu1"]["W"], ((0, 0), (0, HIDK - 2)))
    bm1p = jnp.pad(r1(params["mu1"]["b"]), ((0, 0), (0, HIDK - 2)))
    ws1p = jnp.pad(params["sig1"]["W"], ((0, 0), (0, HIDK - 2)))
    bs1p = jnp.pad(r1(params["sig1"]["b"]), ((0, 0), (0, HIDK - 2)))
    _, _, _, mu, sig = _k4_call(
        x, batch2, params["pool_proj"]["W"], r1(params["pool_proj"]["b"]),
        params["mu0"]["W"], r1(params["mu0"]["b"]), wm1p, bm1p,
        params["sig0"]["W"], r1(params["sig0"]["b"]), ws1p, bs1p)
    return mu[:, :2], sig[:, :2]


# speedup vs baseline: 4.4872x; 1.3955x over previous
import functools
import jax
import jax.numpy as jnp
from jax import lax
from jax.experimental import pallas as pl
from jax.experimental.pallas import tpu as pltpu
from jax.experimental.pallas import tpu_sc as plsc

HIDK = 64
TDIMK = 64

N_NODES = 10240
N_EDGES = 320000
NB_ROWS = 1024
EB_ROWS = 512
SC_CHUNK = 80
ROWS_PER_TILE = N_NODES // 16
EDGES_PER_TILE = N_EDGES // 32
N_CHUNKS = EDGES_PER_TILE // SC_CHUNK


def _silu(x):
    return x * jax.nn.sigmoid(x)


def _ln2(x, g, b):
    mu = jnp.mean(x, axis=-1, keepdims=True)
    var = jnp.var(x, axis=-1, keepdims=True)
    return (x - mu) / jnp.sqrt(var + 1e-5) * g + b


def _k1_body(theta_ref, batch_ref, emb_ref, wn_ref, bn_ref,
             wt0_ref, bt0_ref, wt1_ref, bt1_ref, wtp_ref, btp_ref,
             wr_ref, wc_ref,
             x_ref, xr_ref, xc_ref):
    th = theta_ref[...]
    m = jnp.max(th, axis=-1, keepdims=True)
    e = jnp.exp(th - m)
    sm = e / jnp.sum(e, axis=-1, keepdims=True)
    x = jnp.dot(sm, wn_ref[...], preferred_element_type=jnp.float32) + bn_ref[...]
    emb = emb_ref[...]
    t0 = _silu(jnp.dot(emb, wt0_ref[...], preferred_element_type=jnp.float32) + bt0_ref[...])
    temb = jnp.dot(t0, wt1_ref[...], preferred_element_type=jnp.float32) + bt1_ref[...]
    temb = jnp.dot(temb, wtp_ref[...], preferred_element_type=jnp.float32) + btp_ref[...]
    bb = batch_ref[...]
    oh = (bb == lax.broadcasted_iota(jnp.int32, (1, 32), 1)).astype(jnp.float32)
    x = x + jnp.dot(oh, temb, preferred_element_type=jnp.float32)
    x_ref[...] = x
    xr_ref[...] = jnp.dot(x, wr_ref[...], preferred_element_type=jnp.float32)
    xc_ref[...] = jnp.dot(x, wc_ref[...], preferred_element_type=jnp.float32)


def _k1_call(theta_pad, batch2, emb, wn_pad, bn, wt0, bt0, wt1, bt1, wtp, btp, wr, wc):
    n = theta_pad.shape[0]
    grid = (n // NB_ROWS,)
    full = lambda shp: pl.BlockSpec(shp, lambda i: (0,) * len(shp))
    return pl.pallas_call(
        _k1_body,
        grid=grid,
        in_specs=[
            pl.BlockSpec((NB_ROWS, 128), lambda i: (i, 0)),
            pl.BlockSpec((NB_ROWS, 1), lambda i: (i, 0)),
            full((32, TDIMK)),
            full((128, HIDK)), full((1, HIDK)),
            full((TDIMK, TDIMK)), full((1, TDIMK)),
            full((TDIMK, TDIMK)), full((1, TDIMK)),
            full((TDIMK, HIDK)), full((1, HIDK)),
            full((HIDK, HIDK)), full((HIDK, HIDK)),
        ],
        out_specs=[pl.BlockSpec((NB_ROWS, HIDK), lambda i: (i, 0))] * 3,
        out_shape=[jax.ShapeDtypeStruct((n, HIDK), jnp.float32)] * 3,
    )(theta_pad, batch2, emb, wn_pad, bn, wt0, bt0, wt1, bt1, wtp, btp, wr, wc)


def _pre_kernel_body(pos_hbm, row_hbm, col_hbm, ev_out, deg_out,
                     pos_sh, deg_sh, idx_r, idx_c, pr, pc, evb, onesb, stage,
                     sem_a, sem_b):
    c = lax.axis_index("c")
    s = lax.axis_index("s")
    wid = c * 16 + s
    r0 = s * ROWS_PER_TILE

    def zbody(i, _):
        stage[i, :] = jnp.zeros((16,), jnp.float32)
        return 0
    lax.fori_loop(0, ROWS_PER_TILE, zbody, 0)
    pltpu.sync_copy(stage, deg_sh.at[pl.ds(r0, ROWS_PER_TILE)])
    pltpu.sync_copy(pos_hbm.at[pl.ds(r0, ROWS_PER_TILE)], stage)
    pltpu.sync_copy(stage, pos_sh.at[pl.ds(r0, ROWS_PER_TILE)])

    def obody(i, _):
        onesb[i, :] = jnp.full((16,), 1.0, jnp.float32)
        return 0
    lax.fori_loop(0, SC_CHUNK, obody, 0)
    plsc.subcore_barrier()

    base = wid * EDGES_PER_TILE

    def chunk(k, _):
        e0 = base + k * SC_CHUNK
        pltpu.sync_copy(row_hbm.at[pl.ds(e0, SC_CHUNK)], idx_r)
        pltpu.sync_copy(col_hbm.at[pl.ds(e0, SC_CHUNK)], idx_c)
        cp_r = pltpu.async_copy(pos_sh.at[idx_r], pr, sem_a)
        cp_c = pltpu.async_copy(pos_sh.at[idx_c], pc, sem_b)
        cp_r.wait()
        cp_c.wait()

        def ebody(i, _):
            evb[i, :] = pc[i, :] - pr[i, :]
            return 0
        lax.fori_loop(0, SC_CHUNK, ebody, 0)
        pltpu.sync_copy(evb, ev_out.at[pl.ds(e0, SC_CHUNK)])
        pltpu.sync_copy(onesb, deg_sh.at[idx_r], add=True)
        return 0

    lax.fori_loop(0, N_CHUNKS, chunk, 0)
    plsc.subcore_barrier()
    pltpu.sync_copy(deg_sh.at[pl.ds(r0, ROWS_PER_TILE)], stage)
    pltpu.sync_copy(stage, deg_out.at[c, pl.ds(r0, ROWS_PER_TILE)])


def _pre_call(pos_pad, row, col):
    mesh = plsc.VectorSubcoreMesh(core_axis_name="c", subcore_axis_name="s")
    kfn = functools.partial(
        pl.kernel,
        mesh=mesh,
        compiler_params=pltpu.CompilerParams(use_tc_tiling_on_sc=False),
        out_type=[
            jax.ShapeDtypeStruct((N_EDGES, 16), jnp.float32),
            jax.ShapeDtypeStruct((2, N_NODES, 16), jnp.float32),
        ],
        scratch_types=[
            pltpu.VMEM_SHARED((N_NODES, 16), jnp.float32),
            pltpu.VMEM_SHARED((N_NODES, 16), jnp.float32),
            pltpu.VMEM((SC_CHUNK,), jnp.int32),
            pltpu.VMEM((SC_CHUNK,), jnp.int32),
            pltpu.VMEM((SC_CHUNK, 16), jnp.float32),
            pltpu.VMEM((SC_CHUNK, 16), jnp.float32),
            pltpu.VMEM((SC_CHUNK, 16), jnp.float32),
            pltpu.VMEM((SC_CHUNK, 16), jnp.float32),
            pltpu.VMEM((ROWS_PER_TILE, 16), jnp.float32),
            pltpu.SemaphoreType.DMA,
            pltpu.SemaphoreType.DMA,
        ],
    )
    return kfn(_pre_kernel_body)(pos_pad, row, col)


def _k2_body(ev_ref, wd0_ref, bd0_ref, wd1_ref, bd1_ref,
             wr0_ref, br0_ref, wr1_ref, br1_ref,
             g_ref, b_ref, weall_ref, b0all_ref,
             ef0_ref, ef1_ref, ef2_ref, ef3_ref):
    ev = ev_ref[...][:, 0:3]
    d2 = jnp.sum(ev * ev, axis=-1, keepdims=True)
    d = jnp.sqrt(d2)
    dirv = ev / (d + 1e-8)
    h = _silu(d * wd0_ref[...] + bd0_ref[...])
    df = jnp.dot(h, wd1_ref[...], preferred_element_type=jnp.float32) + bd1_ref[...]
    wr0 = wr0_ref[...]
    rh = _silu(dirv[:, 0:1] * wr0[0:1, :] + dirv[:, 1:2] * wr0[1:2, :]
               + dirv[:, 2:3] * wr0[2:3, :] + br0_ref[...])
    rf = jnp.dot(rh, wr1_ref[...], preferred_element_type=jnp.float32) + br1_ref[...]
    ef = jnp.concatenate([df, rf], axis=-1)
    ef = _ln2(ef, g_ref[...], b_ref[...])
    weall = weall_ref[...]
    b0all = b0all_ref[...]
    outs = (ef0_ref, ef1_ref, ef2_ref, ef3_ref)
    for l in range(4):
        outs[l][...] = (jnp.dot(ef, weall[:, l * 64:(l + 1) * 64],
                                preferred_element_type=jnp.float32)
                        + b0all[:, l * 64:(l + 1) * 64])


def _k2_call(ev, wd0, bd0, wd1, bd1, wr0p, br0, wr1, br1, g, b, weall, b0all):
    grid = (N_EDGES // EB_ROWS,)
    full = lambda shp: pl.BlockSpec(shp, lambda i: (0,) * len(shp))
    return pl.pallas_call(
        _k2_body,
        grid=grid,
        in_specs=[
            pl.BlockSpec((EB_ROWS, 16), lambda i: (i, 0)),
            full((1, 32)), full((1, 32)),
            full((32, 32)), full((1, 32)),
            full((8, 32)), full((1, 32)),
            full((32, 32)), full((1, 32)),
            full((1, HIDK)), full((1, HIDK)),
            full((HIDK, 256)), full((1, 256)),
        ],
        out_specs=[pl.BlockSpec((EB_ROWS, HIDK), lambda i: (i, 0))] * 4,
        out_shape=[jax.ShapeDtypeStruct((N_EDGES, HIDK), jnp.float32)] * 4,
    )(ev, wd0, bd0, wd1, bd1, wr0p, br0, wr1, br1, g, b, weall, b0all)


def _sc_layer_body(xr_hbm, xc_hbm, row_hbm, col_hbm, ef_hbm, out_hbm,
                   xr_sh, h_sh,
                   ir0, ic0, is0, ef0, gr0, gc0, hb0,
                   ir1, ic1, is1, ef1, gr1, gc1, hb1,
                   ld0, ld1, ga0, gb0, ga1, gb1, sc0, sc1):
    c = lax.axis_index("c")
    s = lax.axis_index("s")
    wid = c * 16 + s
    r0 = s * ROWS_PER_TILE
    n_stage = ROWS_PER_TILE // SC_CHUNK
    C = SC_CHUNK
    IR = (ir0, ir1)
    IC = (ic0, ic1)
    IS = (is0, is1)
    EFB = (ef0, ef1)
    GR = (gr0, gr1)
    GC = (gc0, gc1)
    HB = (hb0, hb1)
    LD = (ld0, ld1)
    GA = (ga0, ga1)
    GB = (gb0, gb1)
    SC = (sc0, sc1)
    base = wid * EDGES_PER_TILE
    last = N_CHUNKS - 1

    def zbody(i, _):
        for j in range(4):
            ef0[i, pl.ds(j * 16, 16)] = jnp.zeros((16,), jnp.float32)
        return 0
    lax.fori_loop(0, C, zbody, 0)
    for q in range(n_stage):
        pltpu.sync_copy(ef0, h_sh.at[pl.ds(r0 + q * C, C)])
    for q in range(n_stage):
        pltpu.sync_copy(xr_hbm.at[pl.ds(r0 + q * C, C)], gr0)
        pltpu.sync_copy(gr0, xr_sh.at[pl.ds(r0 + q * C, C)])
    plsc.subcore_barrier()

    def loads(k, p):
        kc = jnp.minimum(k, last)
        e0 = base + kc * C
        pltpu.make_async_copy(row_hbm.at[pl.ds(e0, C)], IR[p], LD[p]).start()
        pltpu.make_async_copy(col_hbm.at[pl.ds(e0, C)], IC[p], LD[p]).start()
        pltpu.make_async_copy(ef_hbm.at[pl.ds(e0, C)], EFB[p], LD[p]).start()

    def loads_wait(k, p):
        kc = jnp.minimum(k, last)
        e0 = base + kc * C
        pltpu.make_async_copy(row_hbm.at[pl.ds(e0, C)], IR[p], LD[p]).wait()
        pltpu.make_async_copy(col_hbm.at[pl.ds(e0, C)], IC[p], LD[p]).wait()
        pltpu.make_async_copy(ef_hbm.at[pl.ds(e0, C)], EFB[p], LD[p]).wait()

    def gathers(p):
        pltpu.make_async_copy(xr_sh.at[IR[p]], GR[p], GA[p]).start()
        pltpu.make_async_copy(xc_hbm.at[IC[p]], GC[p], GB[p]).start()

    def gathers_wait(p):
        pltpu.make_async_copy(xr_sh.at[IR[p]], GR[p], GA[p]).wait()
        pltpu.make_async_copy(xc_hbm.at[IC[p]], GC[p], GB[p]).wait()

    def scatter(p):
        pltpu.make_async_copy(HB[p], h_sh.at[IS[p]], SC[p]).start(add=True)

    def scatter_wait(p):
        pltpu.make_async_copy(HB[p], h_sh.at[IS[p]], SC[p]).wait()

    def process(k, p):
        q = 1 - p
        gathers_wait(p)

        @pl.when(k >= 2)
        def _():
            scatter_wait(p)

        gr, gc, efb, hb = GR[p], GC[p], EFB[p], HB[p]

        def ebody(ii, _):
            for u in range(4):
                i = ii * 4 + u
                for j in range(4):
                    sl = pl.ds(j * 16, 16)
                    t = gr[i, sl] + gc[i, sl] + efb[i, sl]
                    hb[i, sl] = t / (1.0 + jnp.exp(-t))
            return 0
        lax.fori_loop(0, C // 4, ebody, 0)
        for q16 in range(C // 16):
            IS[p][pl.ds(q16 * 16, 16)] = IR[p][pl.ds(q16 * 16, 16)]
        scatter(p)
        loads(k + 2, p)
        loads_wait(k + 1, q)
        gathers(q)

    loads(jnp.int32(0), 0)
    loads(jnp.int32(1), 1)
    loads_wait(jnp.int32(0), 0)
    gathers(0)

    def pair(it, _):
        process(2 * it, 0)
        process(2 * it + 1, 1)
        return 0
    lax.fori_loop(0, N_CHUNKS // 2, pair, 0)
    process(jnp.int32(last), 0)

    loads_wait(jnp.int32(last + 2), 0)
    gathers_wait(1)
    scatter_wait(1)
    scatter_wait(0)

    plsc.subcore_barrier()
    for q in range(n_stage):
        pltpu.sync_copy(h_sh.at[pl.ds(r0 + q * C, C)], hb0)
        pltpu.sync_copy(hb0, out_hbm.at[c, pl.ds(r0 + q * C, C)])


def _sc_layer_call(xr, xc, row, col, ef):
    mesh = plsc.VectorSubcoreMesh(core_axis_name="c", subcore_axis_name="s")
    kfn = functools.partial(
        pl.kernel,
        mesh=mesh,
        compiler_params=pltpu.CompilerParams(use_tc_tiling_on_sc=False),
        out_type=jax.ShapeDtypeStruct((2, N_NODES, HIDK), jnp.float32),
        scratch_types=(
            [pltpu.VMEM_SHARED((N_NODES, HIDK), jnp.float32)] * 2
            + ([pltpu.VMEM((SC_CHUNK,), jnp.int32)] * 3
               + [pltpu.VMEM((SC_CHUNK, HIDK), jnp.float32)] * 4) * 2
            + [pltpu.SemaphoreType.DMA] * 8
        ),
    )
    return kfn(_sc_layer_body)(xr, xc, row, col, ef)


def _k3_body(with_next, x_ref, hp0_ref, hp1_ref, dg0_ref, dg1_ref,
             w1_ref, b1_ref, g1_ref, bl1_ref,
             wf0_ref, bf0_ref, wf1_ref, bf1_ref, g2_ref, bl2_ref,
             wrn_ref, wcn_ref,
             x_out, xr_out=None, xc_out=None):
    x = x_ref[...]
    hs = hp0_ref[...] + hp1_ref[...]
    deg = dg0_ref[...][:, 0:1] + dg1_ref[...][:, 0:1]
    m = jnp.dot(hs, w1_ref[...], preferred_element_type=jnp.float32) + b1_ref[...] * deg
    x = _ln2(x + m, g1_ref[...], bl1_ref[...])
    f = _silu(jnp.dot(x, wf0_ref[...], preferred_element_type=jnp.float32) + bf0_ref[...])
    f = jnp.dot(f, wf1_ref[...], preferred_element_type=jnp.float32) + bf1_ref[...]
    x = _ln2(x + f, g2_ref[...], bl2_ref[...])
    x_out[...] = x
    if with_next:
        xr_out[...] = jnp.dot(x, wrn_ref[...], preferred_element_type=jnp.float32)
        xc_out[...] = jnp.dot(x, wcn_ref[...], preferred_element_type=jnp.float32)


def _k3_call(with_next, x, hp0, hp1, dg0, dg1,
             w1, b1, g1, bl1, wf0, bf0, wf1, bf1, g2, bl2, wrn, wcn):
    n = x.shape[0]
    grid = (n // NB_ROWS,)
    full = lambda shp: pl.BlockSpec(shp, lambda i: (0,) * len(shp))
    rowspec = pl.BlockSpec((NB_ROWS, HIDK), lambda i: (i, 0))
    n_out = 3 if with_next else 1
    outs = pl.pallas_call(
        functools.partial(_k3_body, with_next),
        grid=grid,
        in_specs=[
            rowspec, rowspec, rowspec,
            pl.BlockSpec((NB_ROWS, 16), lambda i: (i, 0)),
            pl.BlockSpec((NB_ROWS, 16), lambda i: (i, 0)),
            full((HIDK, HIDK)), full((1, HIDK)), full((1, HIDK)), full((1, HIDK)),
            full((HIDK, 2 * HIDK)), full((1, 2 * HIDK)),
            full((2 * HIDK, HIDK)), full((1, HIDK)),
            full((1, HIDK)), full((1, HIDK)),
            full((HIDK, HIDK)), full((HIDK, HIDK)),
        ],
        out_specs=[rowspec] * n_out,
        out_shape=[jax.ShapeDtypeStruct((n, HIDK), jnp.float32)] * n_out,
    )(x, hp0, hp1, dg0, dg1, w1, b1, g1, bl1, wf0, bf0, wf1, bf1, g2, bl2, wrn, wcn)
    if with_next:
        return outs
    return outs[0], None, None


def _k4_body(x_ref, batch_ref, wpp_ref, bpp_ref,
             wm0_ref, bm0_ref, wm1_ref, bm1_ref,
             ws0_ref, bs0_ref, ws1_ref, bs1_ref,
             macc_ref, mxacc_ref, cacc_ref, mu_ref, sig_ref):
    i = pl.program_id(0)
    nsteps = pl.num_programs(0)

    @pl.when(i == 0)
    def _init():
        macc_ref[...] = jnp.zeros((32, HIDK), jnp.float32)
        mxacc_ref[...] = jnp.full((32, HIDK), -1e30, jnp.float32)
        cacc_ref[...] = jnp.zeros((32, HIDK), jnp.float32)

    xb = x_ref[...]
    bb = batch_ref[...]
    oh = (bb == lax.broadcasted_iota(jnp.int32, (1, 32), 1)).astype(jnp.float32)
    macc_ref[...] += lax.dot_general(oh, xb, (((0,), (0,)), ((), ())),
                                     preferred_element_type=jnp.float32)
    cacc_ref[...] += jnp.broadcast_to(jnp.sum(oh, axis=0)[:, None], (32, HIDK))
    rows = []
    for bnum in range(32):
        msk = (bb == bnum)
        rowmax = jnp.max(jnp.where(msk, xb, -1e30), axis=0)
        rows.append(rowmax[None, :])
    mxacc_ref[...] = jnp.maximum(mxacc_ref[...], jnp.concatenate(rows, axis=0))

    @pl.when(i == nsteps - 1)
    def _final():
        counts = jnp.maximum(cacc_ref[...], 1.0)
        mean = macc_ref[...] / counts
        pooled = jnp.concatenate([mean, mxacc_ref[...]], axis=-1)
        pooled = jnp.dot(pooled, wpp_ref[...], preferred_element_type=jnp.float32) + bpp_ref[...]
        hm = _silu(jnp.dot(pooled, wm0_ref[...], preferred_element_type=jnp.float32) + bm0_ref[...])
        mu_ref[...] = jax.nn.sigmoid(
            jnp.dot(hm, wm1_ref[...], preferred_element_type=jnp.float32) + bm1_ref[...])
        hsg = _silu(jnp.dot(pooled, ws0_ref[...], preferred_element_type=jnp.float32) + bs0_ref[...])
        sig_ref[...] = jax.nn.softplus(
            jnp.dot(hsg, ws1_ref[...], preferred_element_type=jnp.float32) + bs1_ref[...])


def _k4_call(x, batch2, wpp, bpp, wm0, bm0, wm1p, bm1p, ws0, bs0, ws1p, bs1p):
    n = x.shape[0]
    grid = (n // NB_ROWS,)
    full = lambda shp: pl.BlockSpec(shp, lambda i: (0,) * len(shp))
    return pl.pallas_call(
        _k4_body,
        grid=grid,
        in_specs=[
            pl.BlockSpec((NB_ROWS, HIDK), lambda i: (i, 0)),
            pl.BlockSpec((NB_ROWS, 1), lambda i: (i, 0)),
            full((2 * HIDK, HIDK)), full((1, HIDK)),
            full((HIDK, HIDK)), full((1, HIDK)),
            full((HIDK, HIDK)), full((1, HIDK)),
            full((HIDK, HIDK)), full((1, HIDK)),
            full((HIDK, HIDK)), full((1, HIDK)),
        ],
        out_specs=[full((32, HIDK))] * 5,
        out_shape=[jax.ShapeDtypeStruct((32, HIDK), jnp.float32)] * 5,
    )(x, batch2, wpp, bpp, wm0, bm0, wm1p, bm1p, ws0, bs0, ws1p, bs1p)


def kernel(theta_t, pos_t, t, batch, edge_index, params):
    n = theta_t.shape[0]
    npad = N_NODES - n

    r1 = lambda v: v.reshape(1, -1)

    theta_pad = jnp.pad(theta_t, ((0, npad), (0, 128 - theta_t.shape[1])),
                        constant_values=-1e30)
    wn_pad = jnp.pad(params["node_in"]["W"], ((0, 128 - theta_t.shape[1]), (0, 0)))
    batch2 = jnp.pad(batch.astype(jnp.int32), (0, npad),
                     constant_values=99).reshape(N_NODES, 1)
    row = edge_index[0].astype(jnp.int32)
    col = edge_index[1].astype(jnp.int32)
    pos_pad = jnp.pad(pos_t, ((0, npad), (0, 13)))

    half = TDIMK // 2
    inv_freq = 1.0 / (10000.0 ** (jnp.arange(half, dtype=jnp.float32) / half))
    sin_inp = t[:, None] * inv_freq[None, :]
    emb = jnp.concatenate([jnp.sin(sin_inp), jnp.cos(sin_inp)], axis=-1)

    blocks = params["blocks"]
    wr_l = [b["msg0"]["W"][:HIDK] for b in blocks]
    wc_l = [b["msg0"]["W"][HIDK:2 * HIDK] for b in blocks]
    weall = jnp.concatenate([b["msg0"]["W"][2 * HIDK:] for b in blocks], axis=1)
    b0all = jnp.concatenate([b["msg0"]["b"] for b in blocks]).reshape(1, 256)

    x, xr, xc = _k1_call(
        theta_pad, batch2, emb, wn_pad, r1(params["node_in"]["b"]),
        params["tproj0"]["W"], r1(params["tproj0"]["b"]),
        params["tproj1"]["W"], r1(params["tproj1"]["b"]),
        params["time_proj"]["W"], r1(params["time_proj"]["b"]),
        wr_l[0], wc_l[0])

    ev, degp = _pre_call(pos_pad, row, col)
    dg0 = degp[0]
    dg1 = degp[1]

    wr0p = jnp.pad(params["dir0"]["W"], ((0, 5), (0, 0)))
    efs = _k2_call(
        ev, r1(params["dist0"]["W"][0]), r1(params["dist0"]["b"]),
        params["dist1"]["W"], r1(params["dist1"]["b"]),
        wr0p, r1(params["dir0"]["b"]),
        params["dir1"]["W"], r1(params["dir1"]["b"]),
        r1(params["edge_norm"]["g"]), r1(params["edge_norm"]["b"]),
        weall, b0all)

    for l, blk in enumerate(blocks):
        hp = _sc_layer_call(xr, xc, row, col, efs[l])
        with_next = l < len(blocks) - 1
        wrn = wr_l[l + 1] if with_next else wr_l[0]
        wcn = wc_l[l + 1] if with_next else wc_l[0]
        x, xr, xc = _k3_call(
            with_next, x, hp[0], hp[1], dg0, dg1,
            blk["msg1"]["W"], r1(blk["msg1"]["b"]),
            r1(blk["ln1"]["g"]), r1(blk["ln1"]["b"]),
            blk["ffn0"]["W"], r1(blk["ffn0"]["b"]),
            blk["ffn1"]["W"], r1(blk["ffn1"]["b"]),
            r1(blk["ln2"]["g"]), r1(blk["ln2"]["b"]),
            wrn, wcn)

    wm1p = jnp.pad(params["mu1"]["W"], ((0, 0), (0, HIDK - 2)))
    bm1p = jnp.pad(r1(params["mu1"]["b"]), ((0, 0), (0, HIDK - 2)))
    ws1p = jnp.pad(params["sig1"]["W"], ((0, 0), (0, HIDK - 2)))
    bs1p = jnp.pad(r1(params["sig1"]["b"]), ((0, 0), (0, HIDK - 2)))
    _, _, _, mu, sig = _k4_call(
        x, batch2, params["pool_proj"]["W"], r1(params["pool_proj"]["b"]),
        params["mu0"]["W"], r1(params["mu0"]["b"]), wm1p, bm1p,
        params["sig0"]["W"], r1(params["sig0"]["b"]), ws1p, bs1p)
    return mu[:, :2], sig[:, :2]


# speedup vs baseline: 4.6657x; 1.0398x over previous
import functools
import jax
import jax.numpy as jnp
from jax import lax
from jax.experimental import pallas as pl
from jax.experimental.pallas import tpu as pltpu
from jax.experimental.pallas import tpu_sc as plsc

HIDK = 64
TDIMK = 64

N_NODES = 10240
N_EDGES = 320000
NB_ROWS = 1024
EB_ROWS = 512
SC_CHUNK = 80
ROWS_PER_TILE = N_NODES // 16
EDGES_PER_TILE = N_EDGES // 32
N_CHUNKS = EDGES_PER_TILE // SC_CHUNK


def _silu(x):
    return x * jax.nn.sigmoid(x)


def _ln2(x, g, b):
    mu = jnp.mean(x, axis=-1, keepdims=True)
    var = jnp.var(x, axis=-1, keepdims=True)
    return (x - mu) / jnp.sqrt(var + 1e-5) * g + b


def _k1_body(theta_ref, batch_ref, emb_ref, wn_ref, bn_ref,
             wt0_ref, bt0_ref, wt1_ref, bt1_ref, wtp_ref, btp_ref,
             wr_ref, wc_ref,
             x_ref, xr_ref, xc_ref):
    th = theta_ref[...]
    m = jnp.max(th, axis=-1, keepdims=True)
    e = jnp.exp(th - m)
    sm = e / jnp.sum(e, axis=-1, keepdims=True)
    x = jnp.dot(sm, wn_ref[...], preferred_element_type=jnp.float32) + bn_ref[...]
    emb = emb_ref[...]
    t0 = _silu(jnp.dot(emb, wt0_ref[...], preferred_element_type=jnp.float32) + bt0_ref[...])
    temb = jnp.dot(t0, wt1_ref[...], preferred_element_type=jnp.float32) + bt1_ref[...]
    temb = jnp.dot(temb, wtp_ref[...], preferred_element_type=jnp.float32) + btp_ref[...]
    bb = batch_ref[...]
    oh = (bb == lax.broadcasted_iota(jnp.int32, (1, 32), 1)).astype(jnp.float32)
    x = x + jnp.dot(oh, temb, preferred_element_type=jnp.float32)
    x_ref[...] = x
    xr_ref[...] = jnp.dot(x, wr_ref[...], preferred_element_type=jnp.float32)
    xc_ref[...] = jnp.dot(x, wc_ref[...], preferred_element_type=jnp.float32)


def _k1_call(theta_pad, batch2, emb, wn_pad, bn, wt0, bt0, wt1, bt1, wtp, btp, wr, wc):
    n = theta_pad.shape[0]
    grid = (n // NB_ROWS,)
    full = lambda shp: pl.BlockSpec(shp, lambda i: (0,) * len(shp))
    return pl.pallas_call(
        _k1_body,
        grid=grid,
        in_specs=[
            pl.BlockSpec((NB_ROWS, 128), lambda i: (i, 0)),
            pl.BlockSpec((NB_ROWS, 1), lambda i: (i, 0)),
            full((32, TDIMK)),
            full((128, HIDK)), full((1, HIDK)),
            full((TDIMK, TDIMK)), full((1, TDIMK)),
            full((TDIMK, TDIMK)), full((1, TDIMK)),
            full((TDIMK, HIDK)), full((1, HIDK)),
            full((HIDK, HIDK)), full((HIDK, HIDK)),
        ],
        out_specs=[pl.BlockSpec((NB_ROWS, HIDK), lambda i: (i, 0))] * 3,
        out_shape=[jax.ShapeDtypeStruct((n, HIDK), jnp.float32)] * 3,
    )(theta_pad, batch2, emb, wn_pad, bn, wt0, bt0, wt1, bt1, wtp, btp, wr, wc)


def _pre_kernel_body(pos_hbm, row_hbm, col_hbm, ev_out, deg_out,
                     pos_sh, deg_sh, idx_r, idx_c, pr, pc, evb, onesb, stage,
                     sem_a, sem_b):
    c = lax.axis_index("c")
    s = lax.axis_index("s")
    wid = c * 16 + s
    r0 = s * ROWS_PER_TILE

    def zbody(i, _):
        stage[i, :] = jnp.zeros((16,), jnp.float32)
        return 0
    lax.fori_loop(0, ROWS_PER_TILE, zbody, 0)
    pltpu.sync_copy(stage, deg_sh.at[pl.ds(r0, ROWS_PER_TILE)])
    pltpu.sync_copy(pos_hbm.at[pl.ds(r0, ROWS_PER_TILE)], stage)
    pltpu.sync_copy(stage, pos_sh.at[pl.ds(r0, ROWS_PER_TILE)])

    def obody(i, _):
        onesb[i, :] = jnp.full((16,), 1.0, jnp.float32)
        return 0
    lax.fori_loop(0, SC_CHUNK, obody, 0)
    plsc.subcore_barrier()

    base = wid * EDGES_PER_TILE

    def chunk(k, _):
        e0 = base + k * SC_CHUNK
        pltpu.sync_copy(row_hbm.at[pl.ds(e0, SC_CHUNK)], idx_r)
        pltpu.sync_copy(col_hbm.at[pl.ds(e0, SC_CHUNK)], idx_c)
        cp_r = pltpu.async_copy(pos_sh.at[idx_r], pr, sem_a)
        cp_c = pltpu.async_copy(pos_sh.at[idx_c], pc, sem_b)
        cp_r.wait()
        cp_c.wait()

        def ebody(i, _):
            evb[i, :] = pc[i, :] - pr[i, :]
            return 0
        lax.fori_loop(0, SC_CHUNK, ebody, 0)
        pltpu.sync_copy(evb, ev_out.at[pl.ds(e0, SC_CHUNK)])
        pltpu.sync_copy(onesb, deg_sh.at[idx_r], add=True)
        return 0

    lax.fori_loop(0, N_CHUNKS, chunk, 0)
    plsc.subcore_barrier()
    pltpu.sync_copy(deg_sh.at[pl.ds(r0, ROWS_PER_TILE)], stage)
    pltpu.sync_copy(stage, deg_out.at[c, pl.ds(r0, ROWS_PER_TILE)])


def _pre_call(pos_pad, row, col):
    mesh = plsc.VectorSubcoreMesh(core_axis_name="c", subcore_axis_name="s")
    kfn = functools.partial(
        pl.kernel,
        mesh=mesh,
        compiler_params=pltpu.CompilerParams(use_tc_tiling_on_sc=False),
        out_type=[
            jax.ShapeDtypeStruct((N_EDGES, 16), jnp.float32),
            jax.ShapeDtypeStruct((2, N_NODES, 16), jnp.float32),
        ],
        scratch_types=[
            pltpu.VMEM_SHARED((N_NODES, 16), jnp.float32),
            pltpu.VMEM_SHARED((N_NODES, 16), jnp.float32),
            pltpu.VMEM((SC_CHUNK,), jnp.int32),
            pltpu.VMEM((SC_CHUNK,), jnp.int32),
            pltpu.VMEM((SC_CHUNK, 16), jnp.float32),
            pltpu.VMEM((SC_CHUNK, 16), jnp.float32),
            pltpu.VMEM((SC_CHUNK, 16), jnp.float32),
            pltpu.VMEM((SC_CHUNK, 16), jnp.float32),
            pltpu.VMEM((ROWS_PER_TILE, 16), jnp.float32),
            pltpu.SemaphoreType.DMA,
            pltpu.SemaphoreType.DMA,
        ],
    )
    return kfn(_pre_kernel_body)(pos_pad, row, col)


def _k2_body(ev_ref, wd0_ref, bd0_ref, wd1_ref, bd1_ref,
             wr0_ref, br0_ref, wr1_ref, br1_ref,
             g_ref, b_ref, weall_ref, b0all_ref,
             ef01_ref, ef23_ref):
    ev = ev_ref[...][:, 0:3]
    d2 = jnp.sum(ev * ev, axis=-1, keepdims=True)
    d = jnp.sqrt(d2)
    dirv = ev / (d + 1e-8)
    h = _silu(d * wd0_ref[...] + bd0_ref[...])
    df = jnp.dot(h, wd1_ref[...], preferred_element_type=jnp.float32) + bd1_ref[...]
    wr0 = wr0_ref[...]
    rh = _silu(dirv[:, 0:1] * wr0[0:1, :] + dirv[:, 1:2] * wr0[1:2, :]
               + dirv[:, 2:3] * wr0[2:3, :] + br0_ref[...])
    rf = jnp.dot(rh, wr1_ref[...], preferred_element_type=jnp.float32) + br1_ref[...]
    ef = jnp.concatenate([df, rf], axis=-1)
    ef = _ln2(ef, g_ref[...], b_ref[...])
    weall = weall_ref[...]
    b0all = b0all_ref[...]
    efall = jnp.dot(ef, weall, preferred_element_type=jnp.float32) + b0all
    ef01_ref[...] = efall[:, 0:128]
    ef23_ref[...] = efall[:, 128:256]


def _k2_call(ev, wd0, bd0, wd1, bd1, wr0p, br0, wr1, br1, g, b, weall, b0all):
    grid = (N_EDGES // EB_ROWS,)
    full = lambda shp: pl.BlockSpec(shp, lambda i: (0,) * len(shp))
    return pl.pallas_call(
        _k2_body,
        grid=grid,
        in_specs=[
            pl.BlockSpec((EB_ROWS, 16), lambda i: (i, 0)),
            full((1, 32)), full((1, 32)),
            full((32, 32)), full((1, 32)),
            full((8, 32)), full((1, 32)),
            full((32, 32)), full((1, 32)),
            full((1, HIDK)), full((1, HIDK)),
            full((HIDK, 256)), full((1, 256)),
        ],
        out_specs=[pl.BlockSpec((EB_ROWS, 128), lambda i: (i, 0))] * 2,
        out_shape=[jax.ShapeDtypeStruct((N_EDGES, 128), jnp.float32)] * 2,
    )(ev, wd0, bd0, wd1, bd1, wr0p, br0, wr1, br1, g, b, weall, b0all)


def _sc_layer_body(ef_off, xr_hbm, xc_hbm, row_hbm, col_hbm, ef_hbm, out_hbm,
                   xr_sh, h_sh,
                   ir0, ic0, is0, ef0, gr0, gc0, hb0,
                   ir1, ic1, is1, ef1, gr1, gc1, hb1,
                   ld0, ld1, ga0, gb0, ga1, gb1, sc0, sc1):
    c = lax.axis_index("c")
    s = lax.axis_index("s")
    wid = c * 16 + s
    r0 = s * ROWS_PER_TILE
    n_stage = ROWS_PER_TILE // SC_CHUNK
    C = SC_CHUNK
    IR = (ir0, ir1)
    IC = (ic0, ic1)
    IS = (is0, is1)
    EFB = (ef0, ef1)
    GR = (gr0, gr1)
    GC = (gc0, gc1)
    HB = (hb0, hb1)
    LD = (ld0, ld1)
    GA = (ga0, ga1)
    GB = (gb0, gb1)
    SC = (sc0, sc1)
    base = wid * EDGES_PER_TILE
    last = N_CHUNKS - 1

    def zbody(i, _):
        for j in range(4):
            ef0[i, pl.ds(j * 16, 16)] = jnp.zeros((16,), jnp.float32)
        return 0
    lax.fori_loop(0, C, zbody, 0)
    for q in range(n_stage):
        pltpu.sync_copy(ef0, h_sh.at[pl.ds(r0 + q * C, C)])
    for q in range(n_stage):
        pltpu.sync_copy(xr_hbm.at[pl.ds(r0 + q * C, C)], gr0)
        pltpu.sync_copy(gr0, xr_sh.at[pl.ds(r0 + q * C, C)])
    plsc.subcore_barrier()

    def loads(k, p):
        kc = jnp.minimum(k, last)
        e0 = base + kc * C
        pltpu.make_async_copy(row_hbm.at[pl.ds(e0, C)], IR[p], LD[p]).start()
        pltpu.make_async_copy(col_hbm.at[pl.ds(e0, C)], IC[p], LD[p]).start()
        pltpu.make_async_copy(ef_hbm.at[pl.ds(e0, C), pl.ds(ef_off, HIDK)],
                              EFB[p], LD[p]).start()

    def loads_wait(k, p):
        kc = jnp.minimum(k, last)
        e0 = base + kc * C
        pltpu.make_async_copy(row_hbm.at[pl.ds(e0, C)], IR[p], LD[p]).wait()
        pltpu.make_async_copy(col_hbm.at[pl.ds(e0, C)], IC[p], LD[p]).wait()
        pltpu.make_async_copy(ef_hbm.at[pl.ds(e0, C), pl.ds(ef_off, HIDK)],
                              EFB[p], LD[p]).wait()

    def gathers(p):
        pltpu.make_async_copy(xr_sh.at[IR[p]], GR[p], GA[p]).start()
        pltpu.make_async_copy(xc_hbm.at[IC[p]], GC[p], GB[p]).start()

    def gathers_wait(p):
        pltpu.make_async_copy(xr_sh.at[IR[p]], GR[p], GA[p]).wait()
        pltpu.make_async_copy(xc_hbm.at[IC[p]], GC[p], GB[p]).wait()

    def scatter(p):
        pltpu.make_async_copy(HB[p], h_sh.at[IS[p]], SC[p]).start(add=True)

    def scatter_wait(p):
        pltpu.make_async_copy(HB[p], h_sh.at[IS[p]], SC[p]).wait()

    def process(k, p):
        q = 1 - p
        gathers_wait(p)

        @pl.when(k >= 2)
        def _():
            scatter_wait(p)

        gr, gc, efb, hb = GR[p], GC[p], EFB[p], HB[p]

        def ebody(ii, _):
            for u in range(4):
                i = ii * 4 + u
                for j in range(4):
                    sl = pl.ds(j * 16, 16)
                    t = gr[i, sl] + gc[i, sl] + efb[i, sl]
                    hb[i, sl] = t / (1.0 + jnp.exp(-t))
            return 0
        lax.fori_loop(0, C // 4, ebody, 0)
        for q16 in range(C // 16):
            IS[p][pl.ds(q16 * 16, 16)] = IR[p][pl.ds(q16 * 16, 16)]
        scatter(p)
        loads(k + 2, p)
        loads_wait(k + 1, q)
        gathers(q)

    loads(jnp.int32(0), 0)
    loads(jnp.int32(1), 1)
    loads_wait(jnp.int32(0), 0)
    gathers(0)

    def pair(it, _):
        process(2 * it, 0)
        process(2 * it + 1, 1)
        return 0
    lax.fori_loop(0, N_CHUNKS // 2, pair, 0)
    process(jnp.int32(last), 0)

    loads_wait(jnp.int32(last + 2), 0)
    gathers_wait(1)
    scatter_wait(1)
    scatter_wait(0)

    plsc.subcore_barrier()
    for q in range(n_stage):
        pltpu.sync_copy(h_sh.at[pl.ds(r0 + q * C, C)], hb0)
        pltpu.sync_copy(hb0, out_hbm.at[c, pl.ds(r0 + q * C, C)])


def _sc_layer_call(xr, xc, row, col, ef, ef_off):
    mesh = plsc.VectorSubcoreMesh(core_axis_name="c", subcore_axis_name="s")
    kfn = functools.partial(
        pl.kernel,
        mesh=mesh,
        compiler_params=pltpu.CompilerParams(use_tc_tiling_on_sc=False),
        out_type=jax.ShapeDtypeStruct((2, N_NODES, HIDK), jnp.float32),
        scratch_types=(
            [pltpu.VMEM_SHARED((N_NODES, HIDK), jnp.float32)] * 2
            + ([pltpu.VMEM((SC_CHUNK,), jnp.int32)] * 3
               + [pltpu.VMEM((SC_CHUNK, HIDK), jnp.float32)] * 4) * 2
            + [pltpu.SemaphoreType.DMA] * 8
        ),
    )
    return kfn(functools.partial(_sc_layer_body, ef_off))(xr, xc, row, col, ef)


def _k3_body(with_next, x_ref, hp0_ref, hp1_ref, dg0_ref, dg1_ref,
             w1_ref, b1_ref, g1_ref, bl1_ref,
             wf0_ref, bf0_ref, wf1_ref, bf1_ref, g2_ref, bl2_ref,
             wrn_ref, wcn_ref,
             x_out, xr_out=None, xc_out=None):
    x = x_ref[...]
    hs = hp0_ref[...] + hp1_ref[...]
    deg = dg0_ref[...][:, 0:1] + dg1_ref[...][:, 0:1]
    m = jnp.dot(hs, w1_ref[...], preferred_element_type=jnp.float32) + b1_ref[...] * deg
    x = _ln2(x + m, g1_ref[...], bl1_ref[...])
    f = _silu(jnp.dot(x, wf0_ref[...], preferred_element_type=jnp.float32) + bf0_ref[...])
    f = jnp.dot(f, wf1_ref[...], preferred_element_type=jnp.float32) + bf1_ref[...]
    x = _ln2(x + f, g2_ref[...], bl2_ref[...])
    x_out[...] = x
    if with_next:
        xr_out[...] = jnp.dot(x, wrn_ref[...], preferred_element_type=jnp.float32)
        xc_out[...] = jnp.dot(x, wcn_ref[...], preferred_element_type=jnp.float32)


def _k3_call(with_next, x, hp0, hp1, dg0, dg1,
             w1, b1, g1, bl1, wf0, bf0, wf1, bf1, g2, bl2, wrn, wcn):
    n = x.shape[0]
    grid = (n // NB_ROWS,)
    full = lambda shp: pl.BlockSpec(shp, lambda i: (0,) * len(shp))
    rowspec = pl.BlockSpec((NB_ROWS, HIDK), lambda i: (i, 0))
    n_out = 3 if with_next else 1
    outs = pl.pallas_call(
        functools.partial(_k3_body, with_next),
        grid=grid,
        in_specs=[
            rowspec, rowspec, rowspec,
            pl.BlockSpec((NB_ROWS, 16), lambda i: (i, 0)),
            pl.BlockSpec((NB_ROWS, 16), lambda i: (i, 0)),
            full((HIDK, HIDK)), full((1, HIDK)), full((1, HIDK)), full((1, HIDK)),
            full((HIDK, 2 * HIDK)), full((1, 2 * HIDK)),
            full((2 * HIDK, HIDK)), full((1, HIDK)),
            full((1, HIDK)), full((1, HIDK)),
            full((HIDK, HIDK)), full((HIDK, HIDK)),
        ],
        out_specs=[rowspec] * n_out,
        out_shape=[jax.ShapeDtypeStruct((n, HIDK), jnp.float32)] * n_out,
    )(x, hp0, hp1, dg0, dg1, w1, b1, g1, bl1, wf0, bf0, wf1, bf1, g2, bl2, wrn, wcn)
    if with_next:
        return outs
    return outs[0], None, None


def _k4_body(x_ref, batch_ref, wpp_ref, bpp_ref,
             wm0_ref, bm0_ref, wm1_ref, bm1_ref,
             ws0_ref, bs0_ref, ws1_ref, bs1_ref,
             macc_ref, mxacc_ref, cacc_ref, mu_ref, sig_ref):
    i = pl.program_id(0)
    nsteps = pl.num_programs(0)

    @pl.when(i == 0)
    def _init():
        macc_ref[...] = jnp.zeros((32, HIDK), jnp.float32)
        mxacc_ref[...] = jnp.full((32, HIDK), -1e30, jnp.float32)
        cacc_ref[...] = jnp.zeros((32, HIDK), jnp.float32)

    xb = x_ref[...]
    bb = batch_ref[...]
    oh = (bb == lax.broadcasted_iota(jnp.int32, (1, 32), 1)).astype(jnp.float32)
    macc_ref[...] += lax.dot_general(oh, xb, (((0,), (0,)), ((), ())),
                                     preferred_element_type=jnp.float32)
    cacc_ref[...] += jnp.broadcast_to(jnp.sum(oh, axis=0)[:, None], (32, HIDK))
    rows = []
    for bnum in range(32):
        msk = (bb == bnum)
        rowmax = jnp.max(jnp.where(msk, xb, -1e30), axis=0)
        rows.append(rowmax[None, :])
    mxacc_ref[...] = jnp.maximum(mxacc_ref[...], jnp.concatenate(rows, axis=0))

    @pl.when(i == nsteps - 1)
    def _final():
        counts = jnp.maximum(cacc_ref[...], 1.0)
        mean = macc_ref[...] / counts
        pooled = jnp.concatenate([mean, mxacc_ref[...]], axis=-1)
        pooled = jnp.dot(pooled, wpp_ref[...], preferred_element_type=jnp.float32) + bpp_ref[...]
        hm = _silu(jnp.dot(pooled, wm0_ref[...], preferred_element_type=jnp.float32) + bm0_ref[...])
        mu_ref[...] = jax.nn.sigmoid(
            jnp.dot(hm, wm1_ref[...], preferred_element_type=jnp.float32) + bm1_ref[...])
        hsg = _silu(jnp.dot(pooled, ws0_ref[...], preferred_element_type=jnp.float32) + bs0_ref[...])
        sig_ref[...] = jax.nn.softplus(
            jnp.dot(hsg, ws1_ref[...], preferred_element_type=jnp.float32) + bs1_ref[...])


def _k4_call(x, batch2, wpp, bpp, wm0, bm0, wm1p, bm1p, ws0, bs0, ws1p, bs1p):
    n = x.shape[0]
    grid = (n // NB_ROWS,)
    full = lambda shp: pl.BlockSpec(shp, lambda i: (0,) * len(shp))
    return pl.pallas_call(
        _k4_body,
        grid=grid,
        in_specs=[
            pl.BlockSpec((NB_ROWS, HIDK), lambda i: (i, 0)),
            pl.BlockSpec((NB_ROWS, 1), lambda i: (i, 0)),
            full((2 * HIDK, HIDK)), full((1, HIDK)),
            full((HIDK, HIDK)), full((1, HIDK)),
            full((HIDK, HIDK)), full((1, HIDK)),
            full((HIDK, HIDK)), full((1, HIDK)),
            full((HIDK, HIDK)), full((1, HIDK)),
        ],
        out_specs=[full((32, HIDK))] * 5,
        out_shape=[jax.ShapeDtypeStruct((32, HIDK), jnp.float32)] * 5,
    )(x, batch2, wpp, bpp, wm0, bm0, wm1p, bm1p, ws0, bs0, ws1p, bs1p)


def kernel(theta_t, pos_t, t, batch, edge_index, params):
    n = theta_t.shape[0]
    npad = N_NODES - n

    r1 = lambda v: v.reshape(1, -1)

    theta_pad = jnp.pad(theta_t, ((0, npad), (0, 128 - theta_t.shape[1])),
                        constant_values=-1e30)
    wn_pad = jnp.pad(params["node_in"]["W"], ((0, 128 - theta_t.shape[1]), (0, 0)))
    batch2 = jnp.pad(batch.astype(jnp.int32), (0, npad),
                     constant_values=99).reshape(N_NODES, 1)
    row = edge_index[0].astype(jnp.int32)
    col = edge_index[1].astype(jnp.int32)
    pos_pad = jnp.pad(pos_t, ((0, npad), (0, 13)))

    half = TDIMK // 2
    inv_freq = 1.0 / (10000.0 ** (jnp.arange(half, dtype=jnp.float32) / half))
    sin_inp = t[:, None] * inv_freq[None, :]
    emb = jnp.concatenate([jnp.sin(sin_inp), jnp.cos(sin_inp)], axis=-1)

    blocks = params["blocks"]
    wr_l = [b["msg0"]["W"][:HIDK] for b in blocks]
    wc_l = [b["msg0"]["W"][HIDK:2 * HIDK] for b in blocks]
    weall = jnp.concatenate([b["msg0"]["W"][2 * HIDK:] for b in blocks], axis=1)
    b0all = jnp.concatenate([b["msg0"]["b"] for b in blocks]).reshape(1, 256)

    x, xr, xc = _k1_call(
        theta_pad, batch2, emb, wn_pad, r1(params["node_in"]["b"]),
        params["tproj0"]["W"], r1(params["tproj0"]["b"]),
        params["tproj1"]["W"], r1(params["tproj1"]["b"]),
        params["time_proj"]["W"], r1(params["time_proj"]["b"]),
        wr_l[0], wc_l[0])

    ev, degp = _pre_call(pos_pad, row, col)
    dg0 = degp[0]
    dg1 = degp[1]

    wr0p = jnp.pad(params["dir0"]["W"], ((0, 5), (0, 0)))
    efs = _k2_call(
        ev, r1(params["dist0"]["W"][0]), r1(params["dist0"]["b"]),
        params["dist1"]["W"], r1(params["dist1"]["b"]),
        wr0p, r1(params["dir0"]["b"]),
        params["dir1"]["W"], r1(params["dir1"]["b"]),
        r1(params["edge_norm"]["g"]), r1(params["edge_norm"]["b"]),
        weall, b0all)

    for l, blk in enumerate(blocks):
        hp = _sc_layer_call(xr, xc, row, col, efs[l // 2], (l % 2) * HIDK)
        with_next = l < len(blocks) - 1
        wrn = wr_l[l + 1] if with_next else wr_l[0]
        wcn = wc_l[l + 1] if with_next else wc_l[0]
        x, xr, xc = _k3_call(
            with_next, x, hp[0], hp[1], dg0, dg1,
            blk["msg1"]["W"], r1(blk["msg1"]["b"]),
            r1(blk["ln1"]["g"]), r1(blk["ln1"]["b"]),
            blk["ffn0"]["W"], r1(blk["ffn0"]["b"]),
            blk["ffn1"]["W"], r1(blk["ffn1"]["b"]),
            r1(blk["ln2"]["g"]), r1(blk["ln2"]["b"]),
            wrn, wcn)

    wm1p = jnp.pad(params["mu1"]["W"], ((0, 0), (0, HIDK - 2)))
    bm1p = jnp.pad(r1(params["mu1"]["b"]), ((0, 0), (0, HIDK - 2)))
    ws1p = jnp.pad(params["sig1"]["W"], ((0, 0), (0, HIDK - 2)))
    bs1p = jnp.pad(r1(params["sig1"]["b"]), ((0, 0), (0, HIDK - 2)))
    _, _, _, mu, sig = _k4_call(
        x, batch2, params["pool_proj"]["W"], r1(params["pool_proj"]["b"]),
        params["mu0"]["W"], r1(params["mu0"]["b"]), wm1p, bm1p,
        params["sig0"]["W"], r1(params["sig0"]["b"]), ws1p, bs1p)
    return mu[:, :2], sig[:, :2]
